# baseline (speedup 1.0000x reference)
import functools

import jax
import jax.numpy as jnp
from jax import lax
from jax.experimental import pallas as pl
from jax.experimental.pallas import tpu as pltpu
from jax.experimental.pallas import tpu_sc as plsc

NC = 2
NS = 16
NW = NC * NS
CHUNK = 128
NBUF = 4
LANE = 16


@functools.lru_cache(maxsize=None)
def _build(n_chunks: int, vocab: int, dim: int):
    assert n_chunks % NBUF == 0
    assert dim % LANE == 0
    n_groups = n_chunks // NBUF
    mesh = plsc.VectorSubcoreMesh(core_axis_name="c", subcore_axis_name="s")

    @functools.partial(
        pl.kernel,
        out_type=jax.ShapeDtypeStruct((NW, n_chunks, CHUNK, dim), jnp.float32),
        mesh=mesh,
        compiler_params=pltpu.CompilerParams(needs_layout_passes=False),
        scratch_types=[
            pltpu.VMEM((vocab, dim), jnp.float32),
            pltpu.VMEM((n_chunks * CHUNK,), jnp.int32),
            [pltpu.VMEM((CHUNK, dim), jnp.float32) for _ in range(NBUF)],
            pltpu.SemaphoreType.DMA((NBUF,)),
        ],
    )
    def emb(table_hbm, aa_hbm, out_hbm, table_v, idx_v, rows, ssem):
        wid = lax.axis_index("s") * NC + lax.axis_index("c")
        pltpu.sync_copy(table_hbm, table_v)
        pltpu.sync_copy(aa_hbm.at[wid], idx_v)

        cols = [lax.iota(jnp.int32, LANE) + c * LANE for c in range(dim // LANE)]

        def fill(j, b):
            base = j * CHUNK

            @plsc.parallel_loop(0, CHUNK, unroll=16)
            def row(k):
                pv = jnp.full((LANE,), base + k, jnp.int32)
                ridx = plsc.load_gather(idx_v, [pv])
                for c in range(dim // LANE):
                    v = plsc.load_gather(table_v, [ridx, cols[c]])
                    rows[b][k, pl.ds(c * LANE, LANE)] = v

        def start_scatter(j, b):
            pltpu.async_copy(rows[b], out_hbm.at[wid, j], ssem.at[b])

        def wait_scatter(b):
            pltpu.make_async_copy(rows[b], out_hbm.at[0, 0], ssem.at[b]).wait()

        for b in range(NBUF):
            fill(b, b)
            start_scatter(b, b)

        def step(t, carry):
            for b in range(NBUF):
                wait_scatter(b)
                j = (t + 1) * NBUF + b
                fill(j, b)
                start_scatter(j, b)
            return carry

        lax.fori_loop(0, n_groups - 1, step, 0)
        for b in range(NBUF):
            wait_scatter(b)

    return emb


def kernel(aa, restype_emb):
    B, S = aa.shape
    V, D = restype_emb.shape
    n = B * S
    assert n % (NW * CHUNK) == 0
    n_chunks = n // (NW * CHUNK)
    aa3 = aa.reshape(NW, n_chunks * CHUNK)
    out = _build(n_chunks, V, D)(restype_emb, aa3)
    return out.reshape(B, S, D)


# speedup vs baseline: 1.4814x; 1.4814x over previous
import functools

import jax
import jax.numpy as jnp
from jax import lax
from jax.experimental import pallas as pl
from jax.experimental.pallas import tpu as pltpu
from jax.experimental.pallas import tpu_sc as plsc

NC = 2
NS = 16
NW = NC * NS
CHUNK = 128
NBUF = 4
LANE = 16


@functools.lru_cache(maxsize=None)
def _build(n_chunks: int, vocab: int, dim: int):
    assert n_chunks % NBUF == 0
    assert dim % LANE == 0
    n_groups = n_chunks // NBUF
    mesh = plsc.VectorSubcoreMesh(core_axis_name="c", subcore_axis_name="s")

    @functools.partial(
        pl.kernel,
        out_type=jax.ShapeDtypeStruct((NW, n_chunks, CHUNK, dim), jnp.float32),
        mesh=mesh,
        compiler_params=pltpu.CompilerParams(needs_layout_passes=False),
        scratch_types=[
            pltpu.VMEM((vocab, dim), jnp.float32),
            pltpu.VMEM((n_chunks * CHUNK,), jnp.int32),
            [pltpu.VMEM((CHUNK, dim), jnp.float32) for _ in range(NBUF)],
            pltpu.SemaphoreType.DMA((NBUF,)),
        ],
    )
    def emb(table_hbm, aa_hbm, out_hbm, table_v, idx_v, rows, ssem):
        wid = lax.axis_index("s") * NC + lax.axis_index("c")
        pltpu.sync_copy(table_hbm, table_v)
        pltpu.sync_copy(aa_hbm.at[wid], idx_v)

        cols = [lax.iota(jnp.int32, LANE) + c * LANE for c in range(dim // LANE)]

        def fill(j, b):
            base = j * CHUNK

            @plsc.parallel_loop(0, CHUNK, unroll=4)
            def row(k):
                pv = jnp.full((LANE,), base + k, jnp.int32)
                ridx = plsc.load_gather(idx_v, [pv])
                for c in range(dim // LANE):
                    v = plsc.load_gather(table_v, [ridx, cols[c]])
                    rows[b][k, pl.ds(c * LANE, LANE)] = v

        def start_scatter(j, b):
            pltpu.async_copy(rows[b], out_hbm.at[wid, j], ssem.at[b])

        def wait_scatter(b):
            pltpu.make_async_copy(rows[b], out_hbm.at[0, 0], ssem.at[b]).wait()

        for b in range(NBUF):
            fill(b, b)
            start_scatter(b, b)

        def step(t, carry):
            for b in range(NBUF):
                wait_scatter(b)
                j = (t + 1) * NBUF + b
                fill(j, b)
                start_scatter(j, b)
            return carry

        lax.fori_loop(0, n_groups - 1, step, 0)
        for b in range(NBUF):
            wait_scatter(b)

    return emb


def kernel(aa, restype_emb):
    B, S = aa.shape
    V, D = restype_emb.shape
    n = B * S
    assert n % (NW * CHUNK) == 0
    n_chunks = n // (NW * CHUNK)
    aa3 = aa.reshape(NW, n_chunks * CHUNK)
    out = _build(n_chunks, V, D)(restype_emb, aa3)
    return out.reshape(B, S, D)


# speedup vs baseline: 1.4970x; 1.0105x over previous
import functools

import jax
import jax.numpy as jnp
from jax import lax
from jax.experimental import pallas as pl
from jax.experimental.pallas import tpu as pltpu
from jax.experimental.pallas import tpu_sc as plsc

NC = 2
NS = 16
NW = NC * NS
CHUNK = 256
NBUF = 2
LANE = 16


@functools.lru_cache(maxsize=None)
def _build(n_chunks: int, vocab: int, dim: int):
    assert n_chunks % NBUF == 0
    assert dim % LANE == 0
    n_groups = n_chunks // NBUF
    mesh = plsc.VectorSubcoreMesh(core_axis_name="c", subcore_axis_name="s")

    @functools.partial(
        pl.kernel,
        out_type=jax.ShapeDtypeStruct((NW, n_chunks, CHUNK, dim), jnp.float32),
        mesh=mesh,
        compiler_params=pltpu.CompilerParams(needs_layout_passes=False),
        scratch_types=[
            pltpu.VMEM((vocab, dim), jnp.float32),
            pltpu.VMEM((n_chunks * CHUNK,), jnp.int32),
            [pltpu.VMEM((CHUNK, dim), jnp.float32) for _ in range(NBUF)],
            pltpu.SemaphoreType.DMA((NBUF,)),
        ],
    )
    def emb(table_hbm, aa_hbm, out_hbm, table_v, idx_v, rows, ssem):
        wid = lax.axis_index("s") * NC + lax.axis_index("c")
        pltpu.sync_copy(table_hbm, table_v)
        pltpu.sync_copy(aa_hbm.at[wid], idx_v)

        cols = [lax.iota(jnp.int32, LANE) + c * LANE for c in range(dim // LANE)]

        def fill(j, b):
            base = j * CHUNK

            @plsc.parallel_loop(0, CHUNK, unroll=4)
            def row(k):
                pv = jnp.full((LANE,), base + k, jnp.int32)
                ridx = plsc.load_gather(idx_v, [pv])
                for c in range(dim // LANE):
                    v = plsc.load_gather(table_v, [ridx, cols[c]])
                    rows[b][k, pl.ds(c * LANE, LANE)] = v

        def start_scatter(j, b):
            pltpu.async_copy(rows[b], out_hbm.at[wid, j], ssem.at[b])

        def wait_scatter(b):
            pltpu.make_async_copy(rows[b], out_hbm.at[0, 0], ssem.at[b]).wait()

        for b in range(NBUF):
            fill(b, b)
            start_scatter(b, b)

        def step(t, carry):
            for b in range(NBUF):
                wait_scatter(b)
                j = (t + 1) * NBUF + b
                fill(j, b)
                start_scatter(j, b)
            return carry

        lax.fori_loop(0, n_groups - 1, step, 0)
        for b in range(NBUF):
            wait_scatter(b)

    return emb


def kernel(aa, restype_emb):
    B, S = aa.shape
    V, D = restype_emb.shape
    n = B * S
    assert n % (NW * CHUNK) == 0
    n_chunks = n // (NW * CHUNK)
    aa3 = aa.reshape(NW, n_chunks * CHUNK)
    out = _build(n_chunks, V, D)(restype_emb, aa3)
    return out.reshape(B, S, D)


# speedup vs baseline: 1.4972x; 1.0001x over previous
import functools

import jax
import jax.numpy as jnp
from jax import lax
from jax.experimental import pallas as pl
from jax.experimental.pallas import tpu as pltpu
from jax.experimental.pallas import tpu_sc as plsc

NC = 2
NS = 16
NW = NC * NS
CHUNK = 320
NBUF = 2
LANE = 16


@functools.lru_cache(maxsize=None)
def _build(n_chunks: int, vocab: int, dim: int):
    assert n_chunks % NBUF == 0
    assert dim % LANE == 0
    n_groups = n_chunks // NBUF
    mesh = plsc.VectorSubcoreMesh(core_axis_name="c", subcore_axis_name="s")

    @functools.partial(
        pl.kernel,
        out_type=jax.ShapeDtypeStruct((NW, n_chunks, CHUNK, dim), jnp.float32),
        mesh=mesh,
        compiler_params=pltpu.CompilerParams(needs_layout_passes=False),
        scratch_types=[
            pltpu.VMEM((vocab, dim), jnp.float32),
            pltpu.VMEM((n_chunks * CHUNK,), jnp.int32),
            [pltpu.VMEM((CHUNK, dim), jnp.float32) for _ in range(NBUF)],
            pltpu.SemaphoreType.DMA((NBUF,)),
        ],
    )
    def emb(table_hbm, aa_hbm, out_hbm, table_v, idx_v, rows, ssem):
        wid = lax.axis_index("s") * NC + lax.axis_index("c")
        pltpu.sync_copy(table_hbm, table_v)
        pltpu.sync_copy(aa_hbm.at[wid], idx_v)

        cols = [lax.iota(jnp.int32, LANE) + c * LANE for c in range(dim // LANE)]

        def fill(j, b):
            base = j * CHUNK

            @plsc.parallel_loop(0, CHUNK, unroll=4)
            def row(k):
                pv = jnp.full((LANE,), base + k, jnp.int32)
                ridx = plsc.load_gather(idx_v, [pv])
                for c in range(dim // LANE):
                    v = plsc.load_gather(table_v, [ridx, cols[c]])
                    rows[b][k, pl.ds(c * LANE, LANE)] = v

        def start_scatter(j, b):
            pltpu.async_copy(rows[b], out_hbm.at[wid, j], ssem.at[b])

        def wait_scatter(b):
            pltpu.make_async_copy(rows[b], out_hbm.at[0, 0], ssem.at[b]).wait()

        for b in range(NBUF):
            fill(b, b)
            start_scatter(b, b)

        def step(t, carry):
            for b in range(NBUF):
                wait_scatter(b)
                j = (t + 1) * NBUF + b
                fill(j, b)
                start_scatter(j, b)
            return carry

        lax.fori_loop(0, n_groups - 1, step, 0)
        for b in range(NBUF):
            wait_scatter(b)

    return emb


def kernel(aa, restype_emb):
    B, S = aa.shape
    V, D = restype_emb.shape
    n = B * S
    assert n % (NW * CHUNK) == 0
    n_chunks = n // (NW * CHUNK)
    aa3 = aa.reshape(NW, n_chunks * CHUNK)
    out = _build(n_chunks, V, D)(restype_emb, aa3)
    return out.reshape(B, S, D)


# speedup vs baseline: 1.4994x; 1.0015x over previous
import functools

import jax
import jax.numpy as jnp
from jax import lax
from jax.experimental import pallas as pl
from jax.experimental.pallas import tpu as pltpu
from jax.experimental.pallas import tpu_sc as plsc

NC = 2
NS = 16
NW = NC * NS
CHUNK = 320
NBUF = 2
LANE = 16


@functools.lru_cache(maxsize=None)
def _build(n_chunks: int, vocab: int, dim: int):
    assert n_chunks % NBUF == 0
    assert dim % LANE == 0
    n_groups = n_chunks // NBUF
    mesh = plsc.VectorSubcoreMesh(core_axis_name="c", subcore_axis_name="s")

    @functools.partial(
        pl.kernel,
        out_type=jax.ShapeDtypeStruct((NW, n_chunks, CHUNK, dim), jnp.float32),
        mesh=mesh,
        compiler_params=pltpu.CompilerParams(needs_layout_passes=False),
        scratch_types=[
            pltpu.VMEM((vocab, dim), jnp.float32),
            pltpu.VMEM((n_chunks * CHUNK,), jnp.int32),
            [pltpu.VMEM((CHUNK, dim), jnp.float32) for _ in range(NBUF)],
            pltpu.SemaphoreType.DMA((NBUF,)),
        ],
    )
    def emb(table_hbm, aa_hbm, out_hbm, table_v, idx_v, rows, ssem):
        wid = lax.axis_index("s") * NC + lax.axis_index("c")
        pltpu.sync_copy(table_hbm, table_v)
        pltpu.sync_copy(aa_hbm.at[wid], idx_v)

        cols = [lax.iota(jnp.int32, LANE) + c * LANE for c in range(dim // LANE)]

        def fill(j, b):
            base = j * CHUNK

            @plsc.parallel_loop(0, CHUNK, unroll=8)
            def row(k):
                pv = jnp.full((LANE,), base + k, jnp.int32)
                ridx = plsc.load_gather(idx_v, [pv])
                for c in range(dim // LANE):
                    v = plsc.load_gather(table_v, [ridx, cols[c]])
                    rows[b][k, pl.ds(c * LANE, LANE)] = v

        def start_scatter(j, b):
            pltpu.async_copy(rows[b], out_hbm.at[wid, j], ssem.at[b])

        def wait_scatter(b):
            pltpu.make_async_copy(rows[b], out_hbm.at[0, 0], ssem.at[b]).wait()

        for b in range(NBUF):
            fill(b, b)
            start_scatter(b, b)

        def step(t, carry):
            for b in range(NBUF):
                wait_scatter(b)
                j = (t + 1) * NBUF + b
                fill(j, b)
                start_scatter(j, b)
            return carry

        lax.fori_loop(0, n_groups - 1, step, 0)
        for b in range(NBUF):
            wait_scatter(b)

    return emb


def kernel(aa, restype_emb):
    B, S = aa.shape
    V, D = restype_emb.shape
    n = B * S
    assert n % (NW * CHUNK) == 0
    n_chunks = n // (NW * CHUNK)
    aa3 = aa.reshape(NW, n_chunks * CHUNK)
    out = _build(n_chunks, V, D)(restype_emb, aa3)
    return out.reshape(B, S, D)
